# baseline (speedup 1.0000x reference)
import functools

import jax
import jax.numpy as jnp
from jax import lax
from jax.experimental import pallas as pl
from jax.experimental.pallas import tpu as pltpu
from jax.experimental.pallas import tpu_sc as plsc

N_EMB = 8192
DIM = 256
N_TOK = 8192
BM = 1024
BK = 1024
NI = N_TOK // BM
NJ = N_EMB // BK
BIG = 2 ** 30
N_ELEM = float(N_TOK * DIM)


def _dist_argmin_body(x_ref, w_ref, idx_ref,
                      rmind_ref, rjidx_ref, x2_ref, w2_ref, m_ref):
    i = pl.program_id(0)
    j = pl.program_id(1)

    @pl.when(j == 0)
    def _():
        x2_ref[...] = jnp.sum(x_ref[...] ** 2, axis=1, keepdims=True)

    @pl.when((i == 0) & (j < NJ))
    def _():
        w2_ref[j] = jnp.sum(w_ref[...] ** 2, axis=1)

    @pl.when(j > 0)
    def _():
        jj = j - 1
        d = (x2_ref[...] + w2_ref[jj][None, :]) - 2.0 * m_ref[...]

        @pl.when(jj == 0)
        def _():
            rmind_ref[...] = d
            rjidx_ref[...] = jnp.zeros((BM, BK), jnp.int32)

        @pl.when(jj > 0)
        def _():
            rm = rmind_ref[...]
            better = d < rm
            rmind_ref[...] = jnp.minimum(rm, d)
            rjidx_ref[...] = jnp.where(better, jj, rjidx_ref[...])

        @pl.when(j == NJ)
        def _():
            rmind = rmind_ref[...]
            dmin = jnp.min(rmind, axis=1)
            lin = (rjidx_ref[...] * BK
                   + lax.broadcasted_iota(jnp.int32, (BM, BK), 1))
            idx_ref[0, 0, :] = jnp.min(
                jnp.where(rmind == dmin[:, None], lin, BIG), axis=1)

    @pl.when(j < NJ)
    def _():
        m_ref[...] = lax.dot_general(x_ref[...], w_ref[...],
                                     (((1,), (1,)), ((), ())),
                                     preferred_element_type=jnp.float32)


def _dist_argmin(x, weight, interpret=False):
    return pl.pallas_call(
        _dist_argmin_body,
        grid=(NI, NJ + 1),
        in_specs=[pl.BlockSpec((BM, DIM), lambda i, j: (i, 0)),
                  pl.BlockSpec((BK, DIM),
                               lambda i, j: (jnp.minimum(j, NJ - 1), 0))],
        out_specs=pl.BlockSpec((1, 1, BM), lambda i, j: (i, 0, 0)),
        out_shape=jax.ShapeDtypeStruct((NI, 1, BM), jnp.int32),
        scratch_shapes=[pltpu.VMEM((BM, BK), jnp.float32),
                        pltpu.VMEM((BM, BK), jnp.int32),
                        pltpu.VMEM((BM, 1), jnp.float32),
                        pltpu.VMEM((NJ, BK), jnp.float32),
                        pltpu.VMEM((BM, BK), jnp.float32)],
        interpret=interpret,
    )(x, weight)


NC = 2
NS = 16
NW = NC * NS
TPW = N_TOK // NW
CH = 128
NCH = TPW // CH

def _gather_hist_body(w_hbm, idx_hbm, zeros_hbm, q_hbm, cnt_hbm,
                      idx_v, rows_v, ones_v, hist_sh, sem):
    c = lax.axis_index("c")
    s = lax.axis_index("s")
    wid = s * NC + c
    pltpu.sync_copy(idx_hbm.at[pl.ds(wid * NCH, NCH)], idx_v)
    cps = [pltpu.async_copy(w_hbm.at[idx_v.at[ch]],
                            rows_v.at[pl.ds(ch * CH, CH)], sem)
           for ch in range(NCH)]
    for cp in cps:
        cp.wait()
    pltpu.sync_copy(rows_v, q_hbm.at[pl.ds(wid * TPW, TPW)])
    for v in range(CH // 16):
        ones_v[pl.ds(v * 16, 16)] = jnp.ones((16,), jnp.float32)

    @pl.when(s == 0)
    def _():
        pltpu.sync_copy(zeros_hbm, hist_sh)

    plsc.subcore_barrier()
    for ch in range(NCH):
        pltpu.sync_copy(ones_v, hist_sh.at[idx_v.at[ch]], add=True)
    plsc.subcore_barrier()

    @pl.when(s == 0)
    def _():
        pltpu.sync_copy(hist_sh, cnt_hbm.at[c])


@functools.cache
def _gather_hist_kernel():
    mesh = plsc.VectorSubcoreMesh(core_axis_name="c", subcore_axis_name="s")
    return pl.kernel(
        _gather_hist_body, mesh=mesh,
        out_type=[jax.ShapeDtypeStruct((N_TOK, DIM), jnp.float32),
                  jax.ShapeDtypeStruct((NC, N_EMB), jnp.float32)],
        scratch_types=[pltpu.VMEM((NCH, CH), jnp.int32),
                       pltpu.VMEM((TPW, DIM), jnp.float32),
                       pltpu.VMEM((CH,), jnp.float32),
                       pltpu.VMEM_SHARED((N_EMB,), jnp.float32),
                       pltpu.SemaphoreType.DMA],
    )


def _finalize_body(x_ref, q_ref, cnt_ref, out_ref, loss_ref, ppl_ref, acc_ref):
    b = pl.program_id(0)
    x = x_ref[...]
    q = q_ref[...]
    diff = q - x
    out_ref[0, :, :] = jnp.transpose(x + diff, (1, 0))
    psum = jnp.sum(diff ** 2)

    @pl.when(b == 0)
    def _():
        acc_ref[0, 0] = 0.0
        p = jnp.sum(cnt_ref[...], axis=0) * (1.0 / N_TOK)
        ent = jnp.sum(p * jnp.log(p + 1e-10))
        ppl_ref[0, 0] = jnp.exp(-ent)

    acc_ref[0, 0] = acc_ref[0, 0] + psum

    @pl.when(b == NI - 1)
    def _():
        t = acc_ref[0, 0] * (1.0 / N_ELEM)
        loss_ref[0, 0] = t + 0.25 * t


def _finalize(x, q, cnt, interpret=False):
    return pl.pallas_call(
        _finalize_body,
        grid=(NI,),
        in_specs=[pl.BlockSpec((BM, DIM), lambda b: (b, 0)),
                  pl.BlockSpec((BM, DIM), lambda b: (b, 0)),
                  pl.BlockSpec((NC, N_EMB), lambda b: (0, 0))],
        out_specs=[pl.BlockSpec((1, DIM, BM), lambda b: (b, 0, 0)),
                   pl.BlockSpec((1, 1), lambda b: (0, 0),
                                memory_space=pltpu.SMEM),
                   pl.BlockSpec((1, 1), lambda b: (0, 0),
                                memory_space=pltpu.SMEM)],
        out_shape=[jax.ShapeDtypeStruct((NI, DIM, BM), jnp.float32),
                   jax.ShapeDtypeStruct((1, 1), jnp.float32),
                   jax.ShapeDtypeStruct((1, 1), jnp.float32)],
        scratch_shapes=[pltpu.SMEM((1, 1), jnp.float32)],
        interpret=interpret,
    )(x, q, cnt)


def kernel(inputs, weight):
    x = jnp.transpose(inputs, (0, 2, 3, 1)).reshape(N_TOK, DIM)
    idx3 = _dist_argmin(x, weight)
    idx2d = idx3.reshape(NW * NCH, CH)
    zeros = jnp.zeros((N_EMB,), jnp.float32)
    q, cnt = _gather_hist_kernel()(weight, idx2d, zeros)
    out3, loss, ppl = _finalize(x, q, cnt)
    return (out3.reshape(8, DIM, 32, 32), loss[0, 0], ppl[0, 0])


# speedup vs baseline: 1.1720x; 1.1720x over previous
import functools

import jax
import jax.numpy as jnp
from jax import lax
from jax.experimental import pallas as pl
from jax.experimental.pallas import tpu as pltpu
from jax.experimental.pallas import tpu_sc as plsc

N_EMB = 8192
DIM = 256
N_TOK = 8192
BM = 1024
BK = 1024
NI = N_TOK // BM
NJ = N_EMB // BK
BIG = 2 ** 30
N_ELEM = float(N_TOK * DIM)


def _dist_argmin_body(x_ref, w_ref, idx_ref,
                      rmind_ref, rjidx_ref, x2_ref, w2_ref, wneg_ref):
    i = pl.program_id(0)
    j = pl.program_id(1)

    @pl.when(j == 0)
    def _():
        x2_ref[...] = jnp.sum(x_ref[...] ** 2, axis=1, keepdims=True)

    @pl.when(i == 0)
    def _():
        wb = w_ref[...]
        w2_ref[j] = jnp.sum(wb ** 2, axis=1)
        wneg_ref[j] = -2.0 * wb

    m2 = lax.dot_general(x_ref[...], wneg_ref[j], (((1,), (1,)), ((), ())),
                         preferred_element_type=jnp.float32)
    d = (x2_ref[...] + w2_ref[j][None, :]) + m2

    @pl.when(j == 0)
    def _():
        rmind_ref[...] = d
        rjidx_ref[...] = jnp.zeros((BM, BK), jnp.int32)

    @pl.when(j > 0)
    def _():
        rm = rmind_ref[...]
        better = d < rm
        rmind_ref[...] = jnp.minimum(rm, d)
        rjidx_ref[...] = jnp.where(better, j, rjidx_ref[...])

    @pl.when(j == NJ - 1)
    def _():
        rmind = rmind_ref[...]
        dmin = jnp.min(rmind, axis=1)
        lin = (rjidx_ref[...] * BK
               + lax.broadcasted_iota(jnp.int32, (BM, BK), 1))
        idx_ref[0, 0, :] = jnp.min(
            jnp.where(rmind == dmin[:, None], lin, BIG), axis=1)


def _dist_argmin(x, weight, interpret=False):
    return pl.pallas_call(
        _dist_argmin_body,
        grid=(NI, NJ),
        in_specs=[pl.BlockSpec((BM, DIM), lambda i, j: (i, 0)),
                  pl.BlockSpec((BK, DIM), lambda i, j: (j, 0))],
        out_specs=pl.BlockSpec((1, 1, BM), lambda i, j: (i, 0, 0)),
        out_shape=jax.ShapeDtypeStruct((NI, 1, BM), jnp.int32),
        scratch_shapes=[pltpu.VMEM((BM, BK), jnp.float32),
                        pltpu.VMEM((BM, BK), jnp.int32),
                        pltpu.VMEM((BM, 1), jnp.float32),
                        pltpu.VMEM((NJ, BK), jnp.float32),
                        pltpu.VMEM((NJ, BK, DIM), jnp.float32)],
        interpret=interpret,
    )(x, weight)


NC = 2
NS = 16
NW = NC * NS
TPW = N_TOK // NW
CH = 128
NCH = TPW // CH

def _gather_hist_body(w_hbm, idx_hbm, zeros_hbm, q_hbm, cnt_hbm,
                      idx_v, rows_v, ones_v, hist_sh, sem):
    c = lax.axis_index("c")
    s = lax.axis_index("s")
    wid = s * NC + c
    pltpu.sync_copy(idx_hbm.at[pl.ds(wid * NCH, NCH)], idx_v)
    cps = [pltpu.async_copy(w_hbm.at[idx_v.at[ch]],
                            rows_v.at[pl.ds(ch * CH, CH)], sem)
           for ch in range(NCH)]
    for cp in cps:
        cp.wait()
    pltpu.sync_copy(rows_v, q_hbm.at[pl.ds(wid * TPW, TPW)])
    for v in range(CH // 16):
        ones_v[pl.ds(v * 16, 16)] = jnp.ones((16,), jnp.float32)

    @pl.when(s == 0)
    def _():
        pltpu.sync_copy(zeros_hbm, hist_sh)

    plsc.subcore_barrier()
    for ch in range(NCH):
        pltpu.sync_copy(ones_v, hist_sh.at[idx_v.at[ch]], add=True)
    plsc.subcore_barrier()

    @pl.when(s == 0)
    def _():
        pltpu.sync_copy(hist_sh, cnt_hbm.at[c])


@functools.cache
def _gather_hist_kernel():
    mesh = plsc.VectorSubcoreMesh(core_axis_name="c", subcore_axis_name="s")
    return pl.kernel(
        _gather_hist_body, mesh=mesh,
        out_type=[jax.ShapeDtypeStruct((N_TOK, DIM), jnp.float32),
                  jax.ShapeDtypeStruct((NC, N_EMB), jnp.float32)],
        scratch_types=[pltpu.VMEM((NCH, CH), jnp.int32),
                       pltpu.VMEM((TPW, DIM), jnp.float32),
                       pltpu.VMEM((CH,), jnp.float32),
                       pltpu.VMEM_SHARED((N_EMB,), jnp.float32),
                       pltpu.SemaphoreType.DMA],
    )


def _finalize_body(x_ref, q_ref, cnt_ref, out_ref, loss_ref, ppl_ref, acc_ref):
    b = pl.program_id(0)
    x = x_ref[...]
    q = q_ref[...]
    diff = q - x
    out_ref[0, :, :] = jnp.transpose(x + diff, (1, 0))
    psum = jnp.sum(diff ** 2)

    @pl.when(b == 0)
    def _():
        acc_ref[0, 0] = 0.0
        p = jnp.sum(cnt_ref[...], axis=0) * (1.0 / N_TOK)
        ent = jnp.sum(p * jnp.log(p + 1e-10))
        ppl_ref[0, 0] = jnp.exp(-ent)

    acc_ref[0, 0] = acc_ref[0, 0] + psum

    @pl.when(b == NI - 1)
    def _():
        t = acc_ref[0, 0] * (1.0 / N_ELEM)
        loss_ref[0, 0] = t + 0.25 * t


def _finalize(x, q, cnt, interpret=False):
    return pl.pallas_call(
        _finalize_body,
        grid=(NI,),
        in_specs=[pl.BlockSpec((BM, DIM), lambda b: (b, 0)),
                  pl.BlockSpec((BM, DIM), lambda b: (b, 0)),
                  pl.BlockSpec((NC, N_EMB), lambda b: (0, 0))],
        out_specs=[pl.BlockSpec((1, DIM, BM), lambda b: (b, 0, 0)),
                   pl.BlockSpec((1, 1), lambda b: (0, 0),
                                memory_space=pltpu.SMEM),
                   pl.BlockSpec((1, 1), lambda b: (0, 0),
                                memory_space=pltpu.SMEM)],
        out_shape=[jax.ShapeDtypeStruct((NI, DIM, BM), jnp.float32),
                   jax.ShapeDtypeStruct((1, 1), jnp.float32),
                   jax.ShapeDtypeStruct((1, 1), jnp.float32)],
        scratch_shapes=[pltpu.SMEM((1, 1), jnp.float32)],
        interpret=interpret,
    )(x, q, cnt)


def kernel(inputs, weight):
    x = jnp.transpose(inputs, (0, 2, 3, 1)).reshape(N_TOK, DIM)
    idx3 = _dist_argmin(x, weight)
    idx2d = idx3.reshape(NW * NCH, CH)
    zeros = jnp.zeros((N_EMB,), jnp.float32)
    q, cnt = _gather_hist_kernel()(weight, idx2d, zeros)
    out3, loss, ppl = _finalize(x, q, cnt)
    return (out3.reshape(8, DIM, 32, 32), loss[0, 0], ppl[0, 0])


# speedup vs baseline: 1.2653x; 1.0796x over previous
import functools

import jax
import jax.numpy as jnp
from jax import lax
from jax.experimental import pallas as pl
from jax.experimental.pallas import tpu as pltpu
from jax.experimental.pallas import tpu_sc as plsc

N_EMB = 8192
DIM = 256
N_TOK = 8192
BM = 1024
BK = 2048
NI = N_TOK // BM
NJ = N_EMB // BK
BIG = 2 ** 30
N_ELEM = float(N_TOK * DIM)


def _dist_argmin_body(x_ref, w_ref, idx_ref,
                      rmind_ref, rjidx_ref, x2_ref, w2_ref, xneg_ref):
    i = pl.program_id(0)
    j = pl.program_id(1)

    @pl.when(j == 0)
    def _():
        xb = x_ref[...]
        x2_ref[...] = jnp.sum(xb ** 2, axis=1, keepdims=True)
        xneg_ref[...] = -2.0 * xb

    @pl.when(i == 0)
    def _():
        w2_ref[j] = jnp.sum(w_ref[...] ** 2, axis=1)

    m2 = lax.dot_general(xneg_ref[...], w_ref[...], (((1,), (1,)), ((), ())),
                         preferred_element_type=jnp.float32)
    d = (x2_ref[...] + w2_ref[j][None, :]) + m2

    @pl.when(j == 0)
    def _():
        rmind_ref[...] = d

    @pl.when(j == 1)
    def _():
        rm = rmind_ref[...]
        better = d < rm
        rmind_ref[...] = jnp.minimum(rm, d)
        rjidx_ref[...] = better.astype(jnp.int32)

    @pl.when(j > 1)
    def _():
        rm = rmind_ref[...]
        better = d < rm
        rmind_ref[...] = jnp.minimum(rm, d)
        rjidx_ref[...] = jnp.where(better, j, rjidx_ref[...])

    @pl.when(j == NJ - 1)
    def _():
        rmind = rmind_ref[...]
        dmin = jnp.min(rmind, axis=1)
        lin = (rjidx_ref[...] * BK
               + lax.broadcasted_iota(jnp.int32, (BM, BK), 1))
        idx_ref[0, 0, :] = jnp.min(
            jnp.where(rmind == dmin[:, None], lin, BIG), axis=1)


def _dist_argmin(x, weight, interpret=False):
    return pl.pallas_call(
        _dist_argmin_body,
        grid=(NI, NJ),
        in_specs=[pl.BlockSpec((BM, DIM), lambda i, j: (i, 0)),
                  pl.BlockSpec((BK, DIM), lambda i, j: (j, 0))],
        out_specs=pl.BlockSpec((1, 1, BM), lambda i, j: (i, 0, 0)),
        out_shape=jax.ShapeDtypeStruct((NI, 1, BM), jnp.int32),
        scratch_shapes=[pltpu.VMEM((BM, BK), jnp.float32),
                        pltpu.VMEM((BM, BK), jnp.int32),
                        pltpu.VMEM((BM, 1), jnp.float32),
                        pltpu.VMEM((NJ, BK), jnp.float32),
                        pltpu.VMEM((BM, DIM), jnp.float32)],
        interpret=interpret,
    )(x, weight)


NC = 2
NS = 16
NW = NC * NS
TPW = N_TOK // NW
CH = 128
NCH = TPW // CH

def _gather_hist_body(w_hbm, idx_hbm, zeros_hbm, q_hbm, cnt_hbm,
                      idx_v, rows_v, ones_v, hist_sh, sem):
    c = lax.axis_index("c")
    s = lax.axis_index("s")
    wid = s * NC + c
    pltpu.sync_copy(idx_hbm.at[pl.ds(wid * NCH, NCH)], idx_v)
    cps = [pltpu.async_copy(w_hbm.at[idx_v.at[ch]],
                            rows_v.at[pl.ds(ch * CH, CH)], sem)
           for ch in range(NCH)]
    for cp in cps:
        cp.wait()
    pltpu.sync_copy(rows_v, q_hbm.at[pl.ds(wid * TPW, TPW)])
    for v in range(CH // 16):
        ones_v[pl.ds(v * 16, 16)] = jnp.ones((16,), jnp.float32)

    @pl.when(s == 0)
    def _():
        pltpu.sync_copy(zeros_hbm, hist_sh)

    plsc.subcore_barrier()
    for ch in range(NCH):
        pltpu.sync_copy(ones_v, hist_sh.at[idx_v.at[ch]], add=True)
    plsc.subcore_barrier()

    @pl.when(s == 0)
    def _():
        pltpu.sync_copy(hist_sh, cnt_hbm.at[c])


@functools.cache
def _gather_hist_kernel():
    mesh = plsc.VectorSubcoreMesh(core_axis_name="c", subcore_axis_name="s")
    return pl.kernel(
        _gather_hist_body, mesh=mesh,
        out_type=[jax.ShapeDtypeStruct((N_TOK, DIM), jnp.float32),
                  jax.ShapeDtypeStruct((NC, N_EMB), jnp.float32)],
        scratch_types=[pltpu.VMEM((NCH, CH), jnp.int32),
                       pltpu.VMEM((TPW, DIM), jnp.float32),
                       pltpu.VMEM((CH,), jnp.float32),
                       pltpu.VMEM_SHARED((N_EMB,), jnp.float32),
                       pltpu.SemaphoreType.DMA],
    )


def _finalize_body(x_ref, q_ref, cnt_ref, out_ref, loss_ref, ppl_ref, acc_ref):
    b = pl.program_id(0)
    x = x_ref[...]
    q = q_ref[...]
    diff = q - x
    out_ref[0, :, :] = jnp.transpose(x + diff, (1, 0))
    psum = jnp.sum(diff ** 2)

    @pl.when(b == 0)
    def _():
        acc_ref[0, 0] = 0.0
        p = jnp.sum(cnt_ref[...], axis=0) * (1.0 / N_TOK)
        ent = jnp.sum(p * jnp.log(p + 1e-10))
        ppl_ref[0, 0] = jnp.exp(-ent)

    acc_ref[0, 0] = acc_ref[0, 0] + psum

    @pl.when(b == NI - 1)
    def _():
        t = acc_ref[0, 0] * (1.0 / N_ELEM)
        loss_ref[0, 0] = t + 0.25 * t


def _finalize(x, q, cnt, interpret=False):
    return pl.pallas_call(
        _finalize_body,
        grid=(NI,),
        in_specs=[pl.BlockSpec((BM, DIM), lambda b: (b, 0)),
                  pl.BlockSpec((BM, DIM), lambda b: (b, 0)),
                  pl.BlockSpec((NC, N_EMB), lambda b: (0, 0))],
        out_specs=[pl.BlockSpec((1, DIM, BM), lambda b: (b, 0, 0)),
                   pl.BlockSpec((1, 1), lambda b: (0, 0),
                                memory_space=pltpu.SMEM),
                   pl.BlockSpec((1, 1), lambda b: (0, 0),
                                memory_space=pltpu.SMEM)],
        out_shape=[jax.ShapeDtypeStruct((NI, DIM, BM), jnp.float32),
                   jax.ShapeDtypeStruct((1, 1), jnp.float32),
                   jax.ShapeDtypeStruct((1, 1), jnp.float32)],
        scratch_shapes=[pltpu.SMEM((1, 1), jnp.float32)],
        interpret=interpret,
    )(x, q, cnt)


def kernel(inputs, weight):
    x = jnp.transpose(inputs, (0, 2, 3, 1)).reshape(N_TOK, DIM)
    idx3 = _dist_argmin(x, weight)
    idx2d = idx3.reshape(NW * NCH, CH)
    zeros = jnp.zeros((N_EMB,), jnp.float32)
    q, cnt = _gather_hist_kernel()(weight, idx2d, zeros)
    out3, loss, ppl = _finalize(x, q, cnt)
    return (out3.reshape(8, DIM, 32, 32), loss[0, 0], ppl[0, 0])
    idx2d = idx3.reshape(NW * NCH, CH)
    zeros = jnp.zeros((N_EMB,), jnp.float32)
    q, cnt = _gather_hist_kernel()(weight, idx2d, zeros)
    out3, loss, ppl = _finalize(x, q, cnt)
    return (out3.reshape(8, DIM, 32, 32), loss[0, 0], ppl[0, 0])
